# baseline (speedup 1.0000x reference)
import jax
import jax.numpy as jnp
from jax import lax
from jax.experimental import pallas as pl
from jax.experimental.pallas import tpu as pltpu
from jax.experimental.pallas import tpu_sc as plsc

B, L = 4096, 200
ITEM_DIM, GENRE_DIM = 64, 32
EMB_DIM = ITEM_DIM + GENRE_DIM
OUT_DIM = 128
TOK = B * L

NH = 4
TOK_H = TOK // NH

NC, NS = 2, 16
NW = NC * NS
PER_W = TOK_H // NW
CHUNK = 128
N_CHUNKS = PER_W // CHUNK
NBUF = 5
SENT = -1

BR = 32
T_BLK = BR * L
GRID_H = TOK_H // T_BLK


def _sc_gather(item_hbm, genre_hbm, iid_hbm, gid_hbm, zrow_hbm,
               out_hbm, iidx_v, gidx_v, *scr):
    wid = lax.axis_index("s") * NC + lax.axis_index("c")
    base = wid * PER_W
    fidx = scr[0:NBUF]
    irows = scr[NBUF:2 * NBUF]
    grows = scr[2 * NBUF:3 * NBUF]
    sem_i = scr[3 * NBUF:4 * NBUF]
    sem_g = scr[4 * NBUF:5 * NBUF]
    sem_f = scr[5 * NBUF]

    pltpu.sync_copy(iid_hbm.at[pl.ds(base, PER_W)], iidx_v)
    pltpu.sync_copy(gid_hbm.at[pl.ds(base, PER_W)], gidx_v)

    def body(t, carry):
        j0 = t * NBUF
        cps = []
        for b in range(NBUF):
            loc = (j0 + b) * CHUNK
            cp_i = pltpu.async_copy(
                item_hbm.at[iidx_v.at[pl.ds(loc, CHUNK)]], irows[b], sem_i[b])
            cp_g = pltpu.async_copy(
                genre_hbm.at[gidx_v.at[pl.ds(loc, CHUNK)]], grows[b], sem_g[b])
            cps.append((cp_i, cp_g))
        for b in range(NBUF):
            loc = (j0 + b) * CHUNK
            off = base + loc
            cp_i, cp_g = cps[b]
            for k in range(CHUNK // 16):
                v = iidx_v[pl.ds(loc + k * 16, 16)]
                fidx[b][pl.ds(k * 16, 16)] = jnp.where(
                    v == 0, jnp.zeros_like(v), jnp.full_like(v, SENT))
            cp_i.wait()
            pltpu.async_copy(
                zrow_hbm.at[plsc.Indices(fidx[b], ignored_value=SENT)],
                irows[b], sem_f).wait()
            cp_g.wait()
            pltpu.sync_copy(
                irows[b], out_hbm.at[pl.ds(off, CHUNK), pl.ds(0, ITEM_DIM)])
            pltpu.sync_copy(
                grows[b],
                out_hbm.at[pl.ds(off, CHUNK), pl.ds(ITEM_DIM, GENRE_DIM)])
        return carry

    lax.fori_loop(0, N_CHUNKS // NBUF, body, 0)


def _tc_body(emb_ref, w_ref, b_ref, pos_ref, out_ref):
    e = emb_ref[:, :EMB_DIM]
    acc = jnp.dot(e, w_ref[...], preferred_element_type=jnp.float32)
    out_ref[...] = acc + b_ref[...] + pos_ref[...]


def _tc_body_chained(emb_ref, w_ref, b_ref, pos_ref, prev_ref, out_ref):
    del prev_ref
    e = emb_ref[:, :EMB_DIM]
    acc = jnp.dot(e, w_ref[...], preferred_element_type=jnp.float32)
    out_ref[...] = acc + b_ref[...] + pos_ref[...]


def kernel(hist_item_id, hist_genre_id, item_table, genre_table, W, b,
           pos_table):
    iid_flat = hist_item_id.reshape(TOK)
    gid_flat = hist_genre_id.reshape(TOK)
    gt = genre_table.at[0].set(0.0)
    zrow = jnp.zeros((8, ITEM_DIM), dtype=jnp.float32)

    mesh = plsc.VectorSubcoreMesh(core_axis_name="c", subcore_axis_name="s")
    sc_gather = pl.kernel(
        _sc_gather,
        out_type=[
            jax.ShapeDtypeStruct((TOK_H, 128), jnp.float32),
        ],
        mesh=mesh,
        scratch_types=[
            pltpu.VMEM((PER_W,), jnp.int32),
            pltpu.VMEM((PER_W,), jnp.int32),
        ] + [pltpu.VMEM((CHUNK,), jnp.int32)] * NBUF
          + [pltpu.VMEM((CHUNK, ITEM_DIM), jnp.float32)] * NBUF
          + [pltpu.VMEM((CHUNK, GENRE_DIM), jnp.float32)] * NBUF
          + [pltpu.SemaphoreType.DMA] * NBUF
          + [pltpu.SemaphoreType.DMA] * NBUF
          + [pltpu.SemaphoreType.DMA],
        compiler_params=pltpu.CompilerParams(use_tc_tiling_on_sc=False),
    )
    embs = []
    for h in range(NH):
        (emb_h,) = sc_gather(
            item_table, gt,
            lax.slice_in_dim(iid_flat, h * TOK_H, (h + 1) * TOK_H),
            lax.slice_in_dim(gid_flat, h * TOK_H, (h + 1) * TOK_H),
            zrow)
        embs.append(emb_h)

    b2 = b.reshape(1, OUT_DIM)
    pos_blk = jnp.tile(pos_table, (BR, 1))

    out = None
    for h in range(NH):
        emb_specs = [
            pl.BlockSpec((T_BLK, 128), lambda i: (i, 0)),
            pl.BlockSpec((EMB_DIM, OUT_DIM), lambda i: (0, 0)),
            pl.BlockSpec((1, OUT_DIM), lambda i: (0, 0)),
            pl.BlockSpec((T_BLK, OUT_DIM), lambda i: (0, 0)),
        ]
        out_spec = pl.BlockSpec((T_BLK, OUT_DIM),
                                lambda i, h=h: (i + h * GRID_H, 0))
        if h == 0:
            out = pl.pallas_call(
                _tc_body,
                grid=(GRID_H,),
                in_specs=emb_specs,
                out_specs=out_spec,
                out_shape=jax.ShapeDtypeStruct((TOK, OUT_DIM), jnp.float32),
            )(embs[h], W, b2, pos_blk)
        else:
            out = pl.pallas_call(
                _tc_body_chained,
                grid=(GRID_H,),
                in_specs=emb_specs + [pl.BlockSpec(memory_space=pl.ANY)],
                out_specs=out_spec,
                out_shape=jax.ShapeDtypeStruct((TOK, OUT_DIM), jnp.float32),
                input_output_aliases={4: 0},
            )(embs[h], W, b2, pos_blk, out)

    return out.reshape(B, L, OUT_DIM)


# speedup vs baseline: 1.0039x; 1.0039x over previous
import jax
import jax.numpy as jnp
from jax import lax
from jax.experimental import pallas as pl
from jax.experimental.pallas import tpu as pltpu
from jax.experimental.pallas import tpu_sc as plsc

B, L = 4096, 200
ITEM_DIM, GENRE_DIM = 64, 32
EMB_DIM = ITEM_DIM + GENRE_DIM
OUT_DIM = 128
TOK = B * L

NH = 2
TOK_H = TOK // NH

NC, NS = 2, 16
NW = NC * NS
PER_W = TOK_H // NW
CHUNK = 128
N_CHUNKS = PER_W // CHUNK
NBUF = 4
SENT = -1

BR = 64
T_BLK = BR * L
GRID_H = TOK_H // T_BLK


def _sc_gather(item_hbm, genre_hbm, iid_hbm, gid_hbm, zrow_hbm,
               out_hbm, iidx_v, gidx_v, *scr):
    wid = lax.axis_index("s") * NC + lax.axis_index("c")
    base = wid * PER_W
    fidx = scr[0:NBUF]
    irows = scr[NBUF:2 * NBUF]
    grows = scr[2 * NBUF:3 * NBUF]
    sem_i = scr[3 * NBUF:4 * NBUF]
    sem_g = scr[4 * NBUF:5 * NBUF]
    sem_f = scr[5 * NBUF]

    pltpu.sync_copy(iid_hbm.at[pl.ds(base, PER_W)], iidx_v)
    pltpu.sync_copy(gid_hbm.at[pl.ds(base, PER_W)], gidx_v)

    def body(t, carry):
        j0 = t * NBUF
        cps = []
        for b in range(NBUF):
            loc = (j0 + b) * CHUNK
            cp_i = pltpu.async_copy(
                item_hbm.at[iidx_v.at[pl.ds(loc, CHUNK)]], irows[b], sem_i[b])
            cp_g = pltpu.async_copy(
                genre_hbm.at[gidx_v.at[pl.ds(loc, CHUNK)]], grows[b], sem_g[b])
            cps.append((cp_i, cp_g))
        for b in range(NBUF):
            loc = (j0 + b) * CHUNK
            off = base + loc
            cp_i, cp_g = cps[b]
            for k in range(CHUNK // 16):
                v = iidx_v[pl.ds(loc + k * 16, 16)]
                fidx[b][pl.ds(k * 16, 16)] = jnp.where(
                    v == 0, jnp.zeros_like(v), jnp.full_like(v, SENT))
            cp_i.wait()
            pltpu.async_copy(
                zrow_hbm.at[plsc.Indices(fidx[b], ignored_value=SENT)],
                irows[b], sem_f).wait()
            cp_g.wait()
            pltpu.sync_copy(
                irows[b], out_hbm.at[pl.ds(off, CHUNK), pl.ds(0, ITEM_DIM)])
            pltpu.sync_copy(
                grows[b],
                out_hbm.at[pl.ds(off, CHUNK), pl.ds(ITEM_DIM, GENRE_DIM)])
        return carry

    lax.fori_loop(0, N_CHUNKS // NBUF, body, 0)


def _tc_body(emb_ref, w_ref, b_ref, pos_ref, out_ref):
    e = emb_ref[:, :EMB_DIM]
    acc = jnp.dot(e, w_ref[...], preferred_element_type=jnp.float32)
    out_ref[...] = acc + b_ref[...] + pos_ref[...]


def _tc_body_chained(emb_ref, w_ref, b_ref, pos_ref, prev_ref, out_ref):
    del prev_ref
    e = emb_ref[:, :EMB_DIM]
    acc = jnp.dot(e, w_ref[...], preferred_element_type=jnp.float32)
    out_ref[...] = acc + b_ref[...] + pos_ref[...]


def kernel(hist_item_id, hist_genre_id, item_table, genre_table, W, b,
           pos_table):
    iid_flat = hist_item_id.reshape(TOK)
    gid_flat = hist_genre_id.reshape(TOK)
    gt = genre_table.at[0].set(0.0)
    zrow = jnp.zeros((8, ITEM_DIM), dtype=jnp.float32)

    mesh = plsc.VectorSubcoreMesh(core_axis_name="c", subcore_axis_name="s")
    sc_gather = pl.kernel(
        _sc_gather,
        out_type=[
            jax.ShapeDtypeStruct((TOK_H, 128), jnp.float32),
        ],
        mesh=mesh,
        scratch_types=[
            pltpu.VMEM((PER_W,), jnp.int32),
            pltpu.VMEM((PER_W,), jnp.int32),
        ] + [pltpu.VMEM((CHUNK,), jnp.int32)] * NBUF
          + [pltpu.VMEM((CHUNK, ITEM_DIM), jnp.float32)] * NBUF
          + [pltpu.VMEM((CHUNK, GENRE_DIM), jnp.float32)] * NBUF
          + [pltpu.SemaphoreType.DMA] * NBUF
          + [pltpu.SemaphoreType.DMA] * NBUF
          + [pltpu.SemaphoreType.DMA],
        compiler_params=pltpu.CompilerParams(use_tc_tiling_on_sc=False),
    )
    embs = []
    for h in range(NH):
        (emb_h,) = sc_gather(
            item_table, gt,
            lax.slice_in_dim(iid_flat, h * TOK_H, (h + 1) * TOK_H),
            lax.slice_in_dim(gid_flat, h * TOK_H, (h + 1) * TOK_H),
            zrow)
        embs.append(emb_h)

    b2 = b.reshape(1, OUT_DIM)
    pos_blk = jnp.tile(pos_table, (BR, 1))

    out = None
    for h in range(NH):
        emb_specs = [
            pl.BlockSpec((T_BLK, 128), lambda i: (i, 0)),
            pl.BlockSpec((EMB_DIM, OUT_DIM), lambda i: (0, 0)),
            pl.BlockSpec((1, OUT_DIM), lambda i: (0, 0)),
            pl.BlockSpec((T_BLK, OUT_DIM), lambda i: (0, 0)),
        ]
        out_spec = pl.BlockSpec((T_BLK, OUT_DIM),
                                lambda i, h=h: (i + h * GRID_H, 0))
        if h == 0:
            out = pl.pallas_call(
                _tc_body,
                grid=(GRID_H,),
                in_specs=emb_specs,
                out_specs=out_spec,
                out_shape=jax.ShapeDtypeStruct((TOK, OUT_DIM), jnp.float32),
            )(embs[h], W, b2, pos_blk)
        else:
            out = pl.pallas_call(
                _tc_body_chained,
                grid=(GRID_H,),
                in_specs=emb_specs + [pl.BlockSpec(memory_space=pl.ANY)],
                out_specs=out_spec,
                out_shape=jax.ShapeDtypeStruct((TOK, OUT_DIM), jnp.float32),
                input_output_aliases={4: 0},
            )(embs[h], W, b2, pos_blk, out)

    return out.reshape(B, L, OUT_DIM)
